# baseline (speedup 1.0000x reference)
import jax
import jax.numpy as jnp
from jax.experimental import pallas as pl
from jax.experimental.pallas import tpu as pltpu

_TOPK = 96
_G = 8
_GB = 4


def _krause_kernel(x_ref, wqa_ref, wqb_ref, wka_ref, wkb_ref,
                   wv_hbm, wo_hbm,
                   bq_ref, bk_ref, bv_ref, bo_ref, ls_ref, out_ref,
                   gates_ref, flag_ref, wv_scr, wo_scr, sem_v, sem_o):
    i = pl.program_id(0)
    nq = pl.num_programs(0) - 1
    gd = _G * (gates_ref.shape[1] // (nq * _G))
    dh = gd // _G
    gb = wv_scr.shape[0]
    dn = (((1,), (1,)), ((), ()))
    x = x_ref[...]

    @pl.when(i < nq)
    def _phase_gate():
        def proj(a_ref, b_ref, bias_ref):
            top = jax.lax.dot_general(x, a_ref[...], dn,
                                      preferred_element_type=jnp.float32)
            bot = jax.lax.dot_general(x, b_ref[...], dn,
                                      preferred_element_type=jnp.float32)
            return jnp.concatenate([top, bot], axis=1) + bias_ref[0]

        q = proj(wqa_ref, wqb_ref, bq_ref)
        k = proj(wka_ref, wkb_ref, bk_ref)

        rows = jax.lax.broadcasted_iota(jnp.int32, (gd, _G), 0)
        cols = jax.lax.broadcasted_iota(jnp.int32, (gd, _G), 1)
        red = jnp.where(rows // dh == cols, 1.0, 0.0)
        dmm = (((1,), (0,)), ((), ()))
        dqk = q - k
        d_real = jax.lax.dot_general(dqk * dqk, red, dmm,
                                     preferred_element_type=jnp.float32)
        d_zero = jax.lax.dot_general(q * q, red, dmm,
                                     preferred_element_type=jnp.float32)

        inv_two_sigma_sq = 0.5 * jnp.exp(-2.0 * ls_ref[0])
        z = (d_real - d_zero) * inv_two_sigma_sq
        gate = jnp.where(d_real < d_zero,
                         1.0 / (1.0 + (_TOPK - 1) * jnp.exp(z)),
                         0.0)

        expand = jnp.where(jax.lax.broadcasted_iota(jnp.int32, (_G, gd), 0)
                           == jax.lax.broadcasted_iota(jnp.int32, (_G, gd), 1) // dh,
                           1.0, 0.0)
        gates_ref[:, pl.ds(i * gd, gd)] = jax.lax.dot_general(
            gate, expand, dmm, preferred_element_type=jnp.float32)

        opened = (jnp.sum(gate) > 0).astype(jnp.int32)
        prev = jnp.where(i == 0, 0, flag_ref[0])
        flag_ref[0] = prev | opened

    @pl.when(i == nq)
    def _phase_value():
        out_ref[...] = jnp.broadcast_to(bo_ref[...], out_ref.shape)

        @pl.when(flag_ref[0] > 0)
        def _open_path():
            def slab(s, _):
                cp_v = pltpu.make_async_copy(
                    wv_hbm.at[pl.ds(s * gb, gb), :], wv_scr, sem_v)
                cp_o = pltpu.make_async_copy(
                    wo_hbm.at[:, pl.ds(s * gb, gb)], wo_scr, sem_o)
                cp_v.start()
                cp_o.start()
                cp_v.wait()
                cp_o.wait()
                v = (jax.lax.dot_general(x, wv_scr[...], dn,
                                         preferred_element_type=jnp.float32)
                     + bv_ref[:, pl.ds(s * gb, gb)])
                y = v * gates_ref[:, pl.ds(s * gb, gb)]
                out_ref[...] += jax.lax.dot_general(
                    y, wo_scr[...], dn, preferred_element_type=jnp.float32)
                return 0

            nslabs = gates_ref.shape[1] // gb
            jax.lax.fori_loop(0, nslabs, slab, 0)


def kernel(x, Wq, bq, Wk, bk, Wv, bv, Wo, bo, log_sigma, current_pos):
    del current_pos
    B, T, E = x.shape
    H = log_sigma.shape[0]
    DH = E // H
    GD = _G * DH
    NQ = H // _G
    GD2 = GD // 2
    GB = _GB * DH

    xf = x.reshape(B, E)
    bq2 = bq.reshape(NQ, 1, GD)
    bk2 = bk.reshape(NQ, 1, GD)
    bv2 = bv.reshape(1, E)
    bo2 = bo.reshape(1, E)
    ls2 = log_sigma.reshape(NQ, 1, _G)

    def slab_ix(i):
        return jnp.minimum(i, NQ - 1)

    half_a = pl.BlockSpec((GD2, E), lambda i: (2 * slab_ix(i), 0))
    half_b = pl.BlockSpec((GD2, E), lambda i: (2 * slab_ix(i) + 1, 0))

    out = pl.pallas_call(
        _krause_kernel,
        grid=(NQ + 1,),
        in_specs=[
            pl.BlockSpec((B, E), lambda i: (0, 0)),
            half_a, half_b,
            half_a, half_b,
            pl.BlockSpec(memory_space=pltpu.MemorySpace.HBM),
            pl.BlockSpec(memory_space=pltpu.MemorySpace.HBM),
            pl.BlockSpec((1, 1, GD), lambda i: (slab_ix(i), 0, 0)),
            pl.BlockSpec((1, 1, GD), lambda i: (slab_ix(i), 0, 0)),
            pl.BlockSpec((1, E), lambda i: (0, 0)),
            pl.BlockSpec((1, E), lambda i: (0, 0)),
            pl.BlockSpec((1, 1, _G), lambda i: (slab_ix(i), 0, 0)),
        ],
        out_specs=pl.BlockSpec((B, E), lambda i: (0, 0)),
        out_shape=jax.ShapeDtypeStruct((B, E), jnp.float32),
        scratch_shapes=[
            pltpu.VMEM((B, E), jnp.float32),
            pltpu.SMEM((1,), jnp.int32),
            pltpu.VMEM((GB, E), jnp.float32),
            pltpu.VMEM((E, GB), jnp.float32),
            pltpu.SemaphoreType.DMA,
            pltpu.SemaphoreType.DMA,
        ],
        compiler_params=pltpu.CompilerParams(
            dimension_semantics=("arbitrary",)),
    )(xf, Wq, Wq, Wk, Wk, Wv, Wo, bq2, bk2, bv2, bo2, ls2)

    return out.reshape(B, 1, E)


# speedup vs baseline: 1.0267x; 1.0267x over previous
import jax
import jax.numpy as jnp
from jax.experimental import pallas as pl
from jax.experimental.pallas import tpu as pltpu

_TOPK = 96
_G = 4
_GB = 4


def _krause_kernel(x_ref, wqa_ref, wqb_ref, wka_ref, wkb_ref,
                   wv_hbm, wo_hbm,
                   bq_ref, bk_ref, bv_ref, bo_ref, ls_ref, out_ref,
                   gates_ref, flag_ref, wv_scr, wo_scr, sem_v, sem_o):
    i = pl.program_id(0)
    nq = pl.num_programs(0) - 1
    gd = _G * (gates_ref.shape[1] // (nq * _G))
    dh = gd // _G
    gb = wv_scr.shape[0]
    dn = (((1,), (1,)), ((), ()))
    x = x_ref[...]

    @pl.when(i < nq)
    def _phase_gate():
        def proj(a_ref, b_ref, bias_ref):
            top = jax.lax.dot_general(x, a_ref[...], dn,
                                      preferred_element_type=jnp.float32)
            bot = jax.lax.dot_general(x, b_ref[...], dn,
                                      preferred_element_type=jnp.float32)
            return jnp.concatenate([top, bot], axis=1) + bias_ref[0]

        q = proj(wqa_ref, wqb_ref, bq_ref)
        k = proj(wka_ref, wkb_ref, bk_ref)

        rows = jax.lax.broadcasted_iota(jnp.int32, (gd, _G), 0)
        cols = jax.lax.broadcasted_iota(jnp.int32, (gd, _G), 1)
        red = jnp.where(rows // dh == cols, 1.0, 0.0)
        dmm = (((1,), (0,)), ((), ()))
        dqk = q - k
        d_real = jax.lax.dot_general(dqk * dqk, red, dmm,
                                     preferred_element_type=jnp.float32)
        d_zero = jax.lax.dot_general(q * q, red, dmm,
                                     preferred_element_type=jnp.float32)

        inv_two_sigma_sq = 0.5 * jnp.exp(-2.0 * ls_ref[0])
        z = (d_real - d_zero) * inv_two_sigma_sq
        gate = jnp.where(d_real < d_zero,
                         1.0 / (1.0 + (_TOPK - 1) * jnp.exp(z)),
                         0.0)

        expand = jnp.where(jax.lax.broadcasted_iota(jnp.int32, (_G, gd), 0)
                           == jax.lax.broadcasted_iota(jnp.int32, (_G, gd), 1) // dh,
                           1.0, 0.0)
        gates_ref[:, pl.ds(i * gd, gd)] = jax.lax.dot_general(
            gate, expand, dmm, preferred_element_type=jnp.float32)

        opened = (jnp.sum(gate) > 0).astype(jnp.int32)
        prev = jnp.where(i == 0, 0, flag_ref[0])
        flag_ref[0] = prev | opened

    @pl.when(i == nq)
    def _phase_value():
        out_ref[...] = jnp.broadcast_to(bo_ref[...], out_ref.shape)

        @pl.when(flag_ref[0] > 0)
        def _open_path():
            def slab(s, _):
                cp_v = pltpu.make_async_copy(
                    wv_hbm.at[pl.ds(s * gb, gb), :], wv_scr, sem_v)
                cp_o = pltpu.make_async_copy(
                    wo_hbm.at[:, pl.ds(s * gb, gb)], wo_scr, sem_o)
                cp_v.start()
                cp_o.start()
                cp_v.wait()
                cp_o.wait()
                v = (jax.lax.dot_general(x, wv_scr[...], dn,
                                         preferred_element_type=jnp.float32)
                     + bv_ref[:, pl.ds(s * gb, gb)])
                y = v * gates_ref[:, pl.ds(s * gb, gb)]
                out_ref[...] += jax.lax.dot_general(
                    y, wo_scr[...], dn, preferred_element_type=jnp.float32)
                return 0

            nslabs = gates_ref.shape[1] // gb
            jax.lax.fori_loop(0, nslabs, slab, 0)


def kernel(x, Wq, bq, Wk, bk, Wv, bv, Wo, bo, log_sigma, current_pos):
    del current_pos
    B, T, E = x.shape
    H = log_sigma.shape[0]
    DH = E // H
    GD = _G * DH
    NQ = H // _G
    GD2 = GD // 2
    GB = _GB * DH

    xf = x.reshape(B, E)
    bq2 = bq.reshape(NQ, 1, GD)
    bk2 = bk.reshape(NQ, 1, GD)
    bv2 = bv.reshape(1, E)
    bo2 = bo.reshape(1, E)
    ls2 = log_sigma.reshape(NQ, 1, _G)

    def slab_ix(i):
        return jnp.minimum(i, NQ - 1)

    half_a = pl.BlockSpec((GD2, E), lambda i: (2 * slab_ix(i), 0))
    half_b = pl.BlockSpec((GD2, E), lambda i: (2 * slab_ix(i) + 1, 0))

    out = pl.pallas_call(
        _krause_kernel,
        grid=(NQ + 1,),
        in_specs=[
            pl.BlockSpec((B, E), lambda i: (0, 0)),
            half_a, half_b,
            half_a, half_b,
            pl.BlockSpec(memory_space=pltpu.MemorySpace.HBM),
            pl.BlockSpec(memory_space=pltpu.MemorySpace.HBM),
            pl.BlockSpec((1, 1, GD), lambda i: (slab_ix(i), 0, 0)),
            pl.BlockSpec((1, 1, GD), lambda i: (slab_ix(i), 0, 0)),
            pl.BlockSpec((1, E), lambda i: (0, 0)),
            pl.BlockSpec((1, E), lambda i: (0, 0)),
            pl.BlockSpec((1, 1, _G), lambda i: (slab_ix(i), 0, 0)),
        ],
        out_specs=pl.BlockSpec((B, E), lambda i: (0, 0)),
        out_shape=jax.ShapeDtypeStruct((B, E), jnp.float32),
        scratch_shapes=[
            pltpu.VMEM((B, E), jnp.float32),
            pltpu.SMEM((1,), jnp.int32),
            pltpu.VMEM((GB, E), jnp.float32),
            pltpu.VMEM((E, GB), jnp.float32),
            pltpu.SemaphoreType.DMA,
            pltpu.SemaphoreType.DMA,
        ],
        compiler_params=pltpu.CompilerParams(
            dimension_semantics=("arbitrary",)),
    )(xf, Wq, Wq, Wk, Wk, Wv, Wo, bq2, bk2, bv2, bo2, ls2)

    return out.reshape(B, 1, E)


# speedup vs baseline: 1.0954x; 1.0670x over previous
import jax
import jax.numpy as jnp
from jax.experimental import pallas as pl
from jax.experimental.pallas import tpu as pltpu

_TOPK = 96
_G = 4


def _krause_kernel(x_ref, wq_hbm, wk_hbm, wv_hbm, wo_hbm,
                   bq_ref, bk_ref, bv_ref, bo_ref, ls_ref, out_ref,
                   gates_ref, flag_ref, wq_scr, wk_scr, wv_scr, wo_scr,
                   sem_q, sem_k, sem_v, sem_o):
    i = pl.program_id(0)
    nq = pl.num_programs(0) - 1
    gd = wv_scr.shape[0]
    gd2 = gd // 2
    dh = gd // _G
    dn = (((1,), (1,)), ((), ()))
    x = x_ref[...]

    def start_pair(s, slot):
        for (hbm, scr, sem) in ((wq_hbm, wq_scr, sem_q), (wk_hbm, wk_scr, sem_k)):
            pltpu.make_async_copy(
                hbm.at[pl.ds(s * gd, gd2), :],
                scr.at[slot, pl.ds(0, gd2), :], sem.at[slot, 0]).start()
            pltpu.make_async_copy(
                hbm.at[pl.ds(s * gd + gd2, gd2), :],
                scr.at[slot, pl.ds(gd2, gd2), :], sem.at[slot, 1]).start()

    def wait_pair(s, slot):
        for (hbm, scr, sem) in ((wq_hbm, wq_scr, sem_q), (wk_hbm, wk_scr, sem_k)):
            pltpu.make_async_copy(
                hbm.at[pl.ds(s * gd, gd2), :],
                scr.at[slot, pl.ds(0, gd2), :], sem.at[slot, 0]).wait()
            pltpu.make_async_copy(
                hbm.at[pl.ds(s * gd + gd2, gd2), :],
                scr.at[slot, pl.ds(gd2, gd2), :], sem.at[slot, 1]).wait()

    @pl.when(i == 0)
    def _prime():
        start_pair(0, 0)

    @pl.when(jnp.logical_and(i < nq, i + 1 < nq))
    def _prefetch():
        start_pair(i + 1, (i + 1) % 2)

    def gate_step(slot):
        wait_pair(i, slot)
        q = jax.lax.dot_general(x, wq_scr[slot], dn,
                                preferred_element_type=jnp.float32) + bq_ref[0]
        k = jax.lax.dot_general(x, wk_scr[slot], dn,
                                preferred_element_type=jnp.float32) + bk_ref[0]

        opened = jnp.zeros((), jnp.int32)
        for hh in range(_G):
            qh = q[:, hh * dh:(hh + 1) * dh]
            kh = k[:, hh * dh:(hh + 1) * dh]
            d_real = jnp.sum((qh - kh) ** 2, axis=1, keepdims=True)
            d_zero = jnp.sum(qh * qh, axis=1, keepdims=True)
            ls = ls_ref[hh, 0, 0]
            inv_two_sigma_sq = 0.5 * jnp.exp(-2.0 * ls)
            z = (d_real - d_zero) * inv_two_sigma_sq
            gate = jnp.where(d_real < d_zero,
                             1.0 / (1.0 + (_TOPK - 1) * jnp.exp(z)),
                             0.0)
            gates_ref[:, pl.ds(i * gd + hh * dh, dh)] = (
                jnp.broadcast_to(gate, (gate.shape[0], dh)))
            n_open = jnp.sum(jnp.where(d_real < d_zero, 1.0, 0.0))
            opened = opened | (n_open > 0).astype(jnp.int32)

        prev = jnp.where(i == 0, 0, flag_ref[0])
        flag_ref[0] = prev | opened

    @pl.when(jnp.logical_and(i < nq, i % 2 == 0))
    def _gate_even():
        gate_step(0)

    @pl.when(jnp.logical_and(i < nq, i % 2 == 1))
    def _gate_odd():
        gate_step(1)

    @pl.when(i == nq)
    def _phase_value():
        out_ref[...] = jnp.broadcast_to(bo_ref[...], out_ref.shape)

        @pl.when(flag_ref[0] > 0)
        def _open_path():
            def slab(s, _):
                cp_v = pltpu.make_async_copy(
                    wv_hbm.at[pl.ds(s * gd, gd), :], wv_scr, sem_v)
                cp_o = pltpu.make_async_copy(
                    wo_hbm.at[:, pl.ds(s * gd, gd)], wo_scr, sem_o)
                cp_v.start()
                cp_o.start()
                cp_v.wait()
                cp_o.wait()
                v = (jax.lax.dot_general(x, wv_scr[...], dn,
                                         preferred_element_type=jnp.float32)
                     + bv_ref[:, pl.ds(s * gd, gd)])
                y = v * gates_ref[:, pl.ds(s * gd, gd)]
                out_ref[...] += jax.lax.dot_general(
                    y, wo_scr[...], dn, preferred_element_type=jnp.float32)
                return 0

            jax.lax.fori_loop(0, nq, slab, 0)


def kernel(x, Wq, bq, Wk, bk, Wv, bv, Wo, bo, log_sigma, current_pos):
    del current_pos
    B, T, E = x.shape
    H = log_sigma.shape[0]
    DH = E // H
    GD = _G * DH
    NQ = H // _G

    xf = x.reshape(B, E)
    bq2 = bq.reshape(NQ, 1, GD)
    bk2 = bk.reshape(NQ, 1, GD)
    bv2 = bv.reshape(1, E)
    bo2 = bo.reshape(1, E)
    ls2 = log_sigma.reshape(H, 1, 1)

    def slab_ix(i):
        return jnp.minimum(i, NQ - 1)

    out = pl.pallas_call(
        _krause_kernel,
        grid=(NQ + 1,),
        in_specs=[
            pl.BlockSpec((B, E), lambda i: (0, 0)),
            pl.BlockSpec(memory_space=pltpu.MemorySpace.HBM),
            pl.BlockSpec(memory_space=pltpu.MemorySpace.HBM),
            pl.BlockSpec(memory_space=pltpu.MemorySpace.HBM),
            pl.BlockSpec(memory_space=pltpu.MemorySpace.HBM),
            pl.BlockSpec((1, 1, GD), lambda i: (slab_ix(i), 0, 0)),
            pl.BlockSpec((1, 1, GD), lambda i: (slab_ix(i), 0, 0)),
            pl.BlockSpec((1, E), lambda i: (0, 0)),
            pl.BlockSpec((1, E), lambda i: (0, 0)),
            pl.BlockSpec((_G, 1, 1), lambda i: (slab_ix(i), 0, 0)),
        ],
        out_specs=pl.BlockSpec((B, E), lambda i: (0, 0)),
        out_shape=jax.ShapeDtypeStruct((B, E), jnp.float32),
        scratch_shapes=[
            pltpu.VMEM((B, E), jnp.float32),
            pltpu.SMEM((1,), jnp.int32),
            pltpu.VMEM((2, GD, E), jnp.float32),
            pltpu.VMEM((2, GD, E), jnp.float32),
            pltpu.VMEM((GD, E), jnp.float32),
            pltpu.VMEM((E, GD), jnp.float32),
            pltpu.SemaphoreType.DMA((2, 2)),
            pltpu.SemaphoreType.DMA((2, 2)),
            pltpu.SemaphoreType.DMA,
            pltpu.SemaphoreType.DMA,
        ],
        compiler_params=pltpu.CompilerParams(
            dimension_semantics=("arbitrary",)),
    )(xf, Wq, Wk, Wv, Wo, bq2, bk2, bv2, bo2, ls2)

    return out.reshape(B, 1, E)
